# baseline (speedup 1.0000x reference)
import jax, jax.numpy as jnp
from jax import lax
from jax.experimental import pallas as pl
from jax.experimental.pallas import tpu as pltpu
from jax.experimental.pallas import tpu_sc as plsc

B, E = 16384, 64
TL, DL = 20, 200
TLP = 24
NW = 32
PW = B // NW
RT = 8
RD = 4
NT = RT * TLP

def _title_chunk(idx_hbm, tbl, out, idx_v, tflat, rows_v, sem, r0):
    pltpu.sync_copy(idx_hbm.at[pl.ds(r0, RT)], idx_v)
    lanes = lax.iota(jnp.int32, 16)
    for q in range(NT // 16):
        row0 = (q * 16) // TLP
        thr = TLP * (row0 + 1) - q * 16
        ge = 1 - lax.shift_right_logical(lanes - thr, 31)
        rr = row0 + ge
        cc = (q * 16 + lanes) - rr * TLP
        over = cc - (TL - 1)
        over_pos = over & (0 - (1 - lax.shift_right_logical(over, 31)))
        cc = cc - over_pos
        tflat[pl.ds(q * 16, 16)] = plsc.load_gather(idx_v, [rr, cc])
    for r in range(RT):
        pltpu.async_copy(tbl.at[tflat.at[pl.ds(TLP * r, TLP)]],
                         rows_v.at[r], sem)
    for r in range(RT):
        pltpu.make_async_copy(tbl.at[tflat.at[pl.ds(TLP * r, TLP)]],
                              rows_v.at[r], sem).wait()
    for r in range(RT):
        pltpu.sync_copy(rows_v.at[r, pl.ds(0, TL)], out.at[r0 + r])

def _desc_chunk(idx_hbm, tbl, out, idx_v, rows_v, sem, r0):
    pltpu.sync_copy(idx_hbm.at[pl.ds(r0, RD)], idx_v)
    for r in range(RD):
        pltpu.async_copy(tbl.at[idx_v.at[r, pl.ds(0, 104)]],
                         rows_v.at[r, pl.ds(0, 104)], sem)
        pltpu.async_copy(tbl.at[idx_v.at[r, pl.ds(104, 96)]],
                         rows_v.at[r, pl.ds(104, 96)], sem)
    for r in range(RD):
        pltpu.make_async_copy(tbl.at[idx_v.at[r, pl.ds(0, 104)]],
                              rows_v.at[r, pl.ds(0, 104)], sem).wait()
        pltpu.make_async_copy(tbl.at[idx_v.at[r, pl.ds(104, 96)]],
                              rows_v.at[r, pl.ds(104, 96)], sem).wait()
    pltpu.sync_copy(rows_v, out.at[pl.ds(r0, RD)])

def _body(t_idx, d_idx, t_tbl, d_tbl, out_t, out_d,
          ti_v, tflat, trows, di_v, drows, sem):
    wid = lax.axis_index("s") * 2 + lax.axis_index("c")
    base = wid * PW


    @pl.loop(0, PW // RT)
    def _t(i):
        _title_chunk(t_idx, t_tbl, out_t, ti_v, tflat, trows, sem,
                     base + i * RT)

    @pl.loop(0, PW // RD)
    def _d(i):
        _desc_chunk(d_idx, d_tbl, out_d, di_v, drows, sem, base + i * RD)

@jax.jit
def _lookup(t_idx, d_idx, t_tbl, d_tbl):
    mesh = plsc.VectorSubcoreMesh(core_axis_name="c", subcore_axis_name="s")
    return pl.kernel(
        _body,
        out_type=(
            jax.ShapeDtypeStruct((B, TL, E), jnp.float32),
            jax.ShapeDtypeStruct((B, DL, E), jnp.float32),
        ),
        mesh=mesh,
        scratch_types=[
            pltpu.VMEM((RT, TL), jnp.int32),
            pltpu.VMEM((NT,), jnp.int32),
            pltpu.VMEM((RT, TLP, E), jnp.float32),
            pltpu.VMEM((RD, DL), jnp.int32),
            pltpu.VMEM((RD, DL, E), jnp.float32),
            pltpu.SemaphoreType.DMA,
        ],
        compiler_params=pltpu.CompilerParams(use_tc_tiling_on_sc=False, needs_layout_passes=False),
    )(t_idx, d_idx, t_tbl, d_tbl)

def kernel(title, description, title_table, description_table):
    return _lookup(title.astype(jnp.int32), description.astype(jnp.int32),
                   title_table, description_table)
